# baseline (speedup 1.0000x reference)
import functools

import jax
import jax.numpy as jnp
from jax import lax
from jax.experimental import pallas as pl
from jax.experimental.pallas import tpu as pltpu
from jax.experimental.pallas import tpu_sc as plsc

N = 10000
E = 320000
NC = 2
NS = 16
NW = NC * NS
K = 128
E_PAD = 327680
NCH_T = E_PAD // NS // K
NCH_W = E_PAD // NW // K
EPT = E_PAD // NS
EPW = E_PAD // NW
D = 5
OFF = D // 2
N_PAD = 10240
RPT = N_PAD // NS
RBLK = 2000
GRID = N // RBLK

_mesh = plsc.VectorSubcoreMesh(core_axis_name="c", subcore_axis_name="s")
_params = pltpu.CompilerParams(use_tc_tiling_on_sc=False)


def _ds(j):
    return pl.ds(pl.multiple_of(j * K, K), K)


def _make_agg(CH, NCH, col_split):
    EPB = NCH * K

    @functools.partial(
        pl.kernel,
        out_type=jax.ShapeDtypeStruct((NC, N_PAD, CH), jnp.float32),
        mesh=_mesh,
        compiler_params=_params,
        scratch_types=[
            pltpu.VMEM((EPB,), jnp.int32),
            pltpu.VMEM((EPB,), jnp.int32),
            [pltpu.VMEM((K, CH), jnp.float32)] * D,
            pltpu.VMEM_SHARED((N_PAD, CH), jnp.float32),
            [pltpu.SemaphoreType.DMA] * D,
            [pltpu.SemaphoreType.DMA] * D,
        ],
    )
    def agg(src_hbm, dst_hbm, table_hbm, zeros_hbm, out_hbm,
            src_v, dst_v, bufs, acc, gsems, ssems):
        cc = lax.axis_index("c")
        ss = lax.axis_index("s")
        wid = ss if col_split else ss * NC + cc

        pltpu.sync_copy(zeros_hbm, acc.at[pl.ds(ss * RPT, RPT)])
        plsc.subcore_barrier()

        pltpu.sync_copy(src_hbm.at[pl.ds(wid * EPB, EPB)], src_v)
        pltpu.sync_copy(dst_hbm.at[pl.ds(wid * EPB, EPB)], dst_v)

        tab = table_hbm.at[cc] if col_split else table_hbm

        def wait_gather(b):
            pltpu.make_async_copy(tab.at[src_v.at[_ds(0)]], bufs[b],
                                  gsems[b]).wait()

        def wait_scatter(b):
            pltpu.make_async_copy(bufs[b], acc.at[dst_v.at[_ds(0)]],
                                  ssems[b]).wait()

        for b in range(OFF):
            pltpu.async_copy(tab.at[src_v.at[_ds(b)]], bufs[b], gsems[b])

        def block(i, carry):
            jb = i * D
            for b in range(D):
                j = jb + b
                b2 = (b + OFF) % D
                wait_gather(b)
                pltpu.async_copy(bufs[b], acc.at[dst_v.at[_ds(j)]], ssems[b],
                                 add=True)
                if b >= D - OFF:
                    wait_scatter(b2)
                else:
                    @pl.when(i > 0)
                    def _():
                        wait_scatter(b2)
                jn = lax.min(j + OFF, NCH - 1)
                pltpu.async_copy(tab.at[src_v.at[_ds(jn)]], bufs[b2],
                                 gsems[b2])
            return carry

        lax.fori_loop(0, NCH // D, block, 0)
        for b in range(OFF, D):
            wait_scatter(b)
        for b in range(OFF):
            wait_gather(b)

        plsc.subcore_barrier()
        pltpu.sync_copy(acc.at[pl.ds(ss * RPT, RPT)],
                        out_hbm.at[cc, pl.ds(ss * RPT, RPT)])

    return agg


_agg1 = _make_agg(64, NCH_T, col_split=True)
_agg2 = _make_agg(64, NCH_W, col_split=False)

_DEGC = 16
_DG = 8


_REAL_LAST = E - (NW - 1) * EPW
_PADN = E_PAD - E


@functools.partial(
    pl.kernel,
    out_type=(
        jax.ShapeDtypeStruct((NC, N_PAD, _DEGC), jnp.float32),
        jax.ShapeDtypeStruct((E_PAD,), jnp.int32),
        jax.ShapeDtypeStruct((E_PAD,), jnp.int32),
    ),
    mesh=_mesh,
    compiler_params=_params,
    scratch_types=[
        pltpu.VMEM((EPW,), jnp.int32),
        pltpu.VMEM((EPW,), jnp.int32),
        pltpu.VMEM((K, _DEGC), jnp.float32),
        pltpu.VMEM_SHARED((N_PAD, _DEGC), jnp.float32),
        pltpu.SemaphoreType.DMA,
    ],
)
def _deg(edge_hbm, pads_hbm, ones_hbm, zeros_hbm,
         out_hbm, src_lin, dst_lin, src_v, dst_v, ones_v, acc, sem):
    cc = lax.axis_index("c")
    ss = lax.axis_index("s")
    wid = ss * NC + cc

    pltpu.sync_copy(zeros_hbm, acc.at[pl.ds(ss * RPT, RPT)])
    plsc.subcore_barrier()

    @pl.when(wid < NW - 1)
    def _():
        pltpu.sync_copy(edge_hbm.at[0, pl.ds(wid * EPW, EPW)], src_v)
        pltpu.sync_copy(edge_hbm.at[1, pl.ds(wid * EPW, EPW)], dst_v)

    @pl.when(wid == NW - 1)
    def _():
        base = (NW - 1) * EPW
        pltpu.sync_copy(edge_hbm.at[0, pl.ds(base, _REAL_LAST)],
                        src_v.at[pl.ds(0, _REAL_LAST)])
        pltpu.sync_copy(edge_hbm.at[1, pl.ds(base, _REAL_LAST)],
                        dst_v.at[pl.ds(0, _REAL_LAST)])
        pltpu.sync_copy(pads_hbm.at[0], src_v.at[pl.ds(_REAL_LAST, _PADN)])
        pltpu.sync_copy(pads_hbm.at[1], dst_v.at[pl.ds(_REAL_LAST, _PADN)])

    pltpu.sync_copy(src_v, src_lin.at[pl.ds(wid * EPW, EPW)])
    pltpu.sync_copy(dst_v, dst_lin.at[pl.ds(wid * EPW, EPW)])

    pltpu.sync_copy(ones_hbm, ones_v)

    def step(i, carry):
        for b in range(_DG):
            pltpu.async_copy(ones_v, acc.at[dst_v.at[_ds(i * _DG + b)]], sem,
                             add=True)
        for b in range(_DG):
            pltpu.make_async_copy(ones_v, acc.at[dst_v.at[_ds(0)]],
                                  sem).wait()
        return carry

    lax.fori_loop(0, NCH_W // _DG, step, 0)

    plsc.subcore_barrier()
    pltpu.sync_copy(acc.at[pl.ds(ss * RPT, RPT)],
                    out_hbm.at[cc, pl.ds(ss * RPT, RPT)])


def _prep_body(degp_ref, x_ref, xp2_ref, dinv_ref):
    deg = degp_ref[0, :, 0:1] + degp_ref[1, :, 0:1] + 1.0
    dinv = lax.rsqrt(deg)
    xp = x_ref[...] * dinv
    xp2_ref[0] = xp[:, :64]
    xp2_ref[1] = xp[:, 64:]
    dinv_ref[...] = dinv


def _prep(degp, x):
    return pl.pallas_call(
        _prep_body,
        grid=(GRID,),
        in_specs=[
            pl.BlockSpec((NC, RBLK, _DEGC), lambda i: (0, i, 0)),
            pl.BlockSpec((RBLK, 128), lambda i: (i, 0)),
        ],
        out_specs=[
            pl.BlockSpec((NC, RBLK, 64), lambda i: (0, i, 0)),
            pl.BlockSpec((RBLK, 1), lambda i: (i, 0)),
        ],
        out_shape=[
            jax.ShapeDtypeStruct((NC, N, 64), jnp.float32),
            jax.ShapeDtypeStruct((N, 1), jnp.float32),
        ],
    )(degp, x)


def _dense_body(p_ref, xp2_ref, dinv_ref, W1_ref, b1_ref, W2_ref, out_ref):
    p = jnp.concatenate([p_ref[0], p_ref[1]], axis=1)
    xp = jnp.concatenate([xp2_ref[0], xp2_ref[1]], axis=1)
    agg = p + xp
    t = (agg * dinv_ref[...]).astype(jnp.bfloat16)
    h1 = jnp.dot(t, W1_ref[...].astype(jnp.bfloat16),
                 preferred_element_type=jnp.float32) + b1_ref[...]
    h1 = jnp.maximum(h1, 0.0).astype(jnp.bfloat16)
    h2 = jnp.dot(h1, W2_ref[...].astype(jnp.bfloat16),
                 preferred_element_type=jnp.float32)
    out_ref[...] = h2 * dinv_ref[...]


def _dense(p, xp2, dinv, W1, b1, W2):
    return pl.pallas_call(
        _dense_body,
        grid=(GRID,),
        in_specs=[
            pl.BlockSpec((NC, RBLK, 64), lambda i: (0, i, 0)),
            pl.BlockSpec((NC, RBLK, 64), lambda i: (0, i, 0)),
            pl.BlockSpec((RBLK, 1), lambda i: (i, 0)),
            pl.BlockSpec((128, 256), lambda i: (0, 0)),
            pl.BlockSpec((1, 256), lambda i: (0, 0)),
            pl.BlockSpec((256, 64), lambda i: (0, 0)),
        ],
        out_specs=pl.BlockSpec((RBLK, 64), lambda i: (i, 0)),
        out_shape=jax.ShapeDtypeStruct((N, 64), jnp.float32),
    )(p, xp2, dinv, W1, b1, W2)


def _final_body(q_ref, hp_ref, dinv_ref, b2_ref, out_ref):
    out_ref[...] = ((q_ref[0] + q_ref[1] + hp_ref[...]) * dinv_ref[...]
                    + b2_ref[...])


def _final(q, hp, dinv, b2):
    return pl.pallas_call(
        _final_body,
        grid=(GRID,),
        in_specs=[
            pl.BlockSpec((NC, RBLK, 64), lambda i: (0, i, 0)),
            pl.BlockSpec((RBLK, 64), lambda i: (i, 0)),
            pl.BlockSpec((RBLK, 1), lambda i: (i, 0)),
            pl.BlockSpec((1, 64), lambda i: (0, 0)),
        ],
        out_specs=pl.BlockSpec((RBLK, 64), lambda i: (i, 0)),
        out_shape=jax.ShapeDtypeStruct((N, 64), jnp.float32),
    )(q, hp, dinv, b2)


@jax.jit
def _run(x, edge_index, W1, b1, W2, b2):
    spread = jnp.arange(_PADN, dtype=jnp.int32) % (N_PAD - N)
    pads = jnp.stack([spread, N + spread])

    degp, src_p, dst_p = _deg(edge_index.astype(jnp.int32), pads,
                              jnp.ones((K, _DEGC), jnp.float32),
                              jnp.zeros((RPT, _DEGC), jnp.float32))
    xp2, dinv = _prep(degp, x)
    p = _agg1(src_p, dst_p, xp2, jnp.zeros((RPT, 64), jnp.float32))
    hp = _dense(p, xp2, dinv, W1, b1.reshape(1, -1), W2)
    q = _agg2(src_p, dst_p, hp, jnp.zeros((RPT, 64), jnp.float32))
    return _final(q, hp, dinv, b2.reshape(1, -1))


def kernel(x, edge_index, W1, b1, W2, b2):
    return _run(x, edge_index, W1, b1, W2, b2)


# speedup vs baseline: 1.1331x; 1.1331x over previous
import functools

import jax
import jax.numpy as jnp
from jax import lax
from jax.experimental import pallas as pl
from jax.experimental.pallas import tpu as pltpu
from jax.experimental.pallas import tpu_sc as plsc

N = 10000
E = 320000
NC = 2
NS = 16
NW = NC * NS
K = 128
E_PAD = 327680
NCH_T = E_PAD // NS // K
NCH_W = E_PAD // NW // K
EPT = E_PAD // NS
EPW = E_PAD // NW
D = 4
OFF = D // 2
N_PAD = 10240
RPT = N_PAD // NS
RBLK = 2000
GRID = N // RBLK

_mesh = plsc.VectorSubcoreMesh(core_axis_name="c", subcore_axis_name="s")
_params = pltpu.CompilerParams(use_tc_tiling_on_sc=False)


def _ds(j):
    return pl.ds(pl.multiple_of(j * K, K), K)


def _make_agg(CH, NCH, col_split, dtype=jnp.float32):
    EPB = NCH * K

    @functools.partial(
        pl.kernel,
        out_type=jax.ShapeDtypeStruct((NC, N_PAD, CH), dtype),
        mesh=_mesh,
        compiler_params=_params,
        scratch_types=[
            pltpu.VMEM((EPB,), jnp.int32),
            pltpu.VMEM((EPB,), jnp.int32),
            [pltpu.VMEM((K, CH), dtype)] * D,
            pltpu.VMEM_SHARED((N_PAD, CH), dtype),
            [pltpu.SemaphoreType.DMA] * D,
            [pltpu.SemaphoreType.DMA] * D,
        ],
    )
    def agg(src_hbm, dst_hbm, table_hbm, zeros_hbm, out_hbm,
            src_v, dst_v, bufs, acc, gsems, ssems):
        cc = lax.axis_index("c")
        ss = lax.axis_index("s")
        wid = ss if col_split else ss * NC + cc

        pltpu.sync_copy(zeros_hbm, acc.at[pl.ds(ss * RPT, RPT)])
        plsc.subcore_barrier()

        pltpu.sync_copy(src_hbm.at[pl.ds(wid * EPB, EPB)], src_v)
        pltpu.sync_copy(dst_hbm.at[pl.ds(wid * EPB, EPB)], dst_v)

        tab = table_hbm.at[cc] if col_split else table_hbm

        def wait_gather(b):
            pltpu.make_async_copy(tab.at[src_v.at[_ds(0)]], bufs[b],
                                  gsems[b]).wait()

        def wait_scatter(b):
            pltpu.make_async_copy(bufs[b], acc.at[dst_v.at[_ds(0)]],
                                  ssems[b]).wait()

        for b in range(OFF):
            pltpu.async_copy(tab.at[src_v.at[_ds(b)]], bufs[b], gsems[b])

        def block(i, carry):
            jb = i * D
            for b in range(D):
                j = jb + b
                b2 = (b + OFF) % D
                wait_gather(b)
                pltpu.async_copy(bufs[b], acc.at[dst_v.at[_ds(j)]], ssems[b],
                                 add=True)
                if b >= D - OFF:
                    wait_scatter(b2)
                else:
                    @pl.when(i > 0)
                    def _():
                        wait_scatter(b2)
                jn = lax.min(j + OFF, NCH - 1)
                pltpu.async_copy(tab.at[src_v.at[_ds(jn)]], bufs[b2],
                                 gsems[b2])
            return carry

        lax.fori_loop(0, NCH // D, block, 0)
        for b in range(OFF, D):
            wait_scatter(b)
        for b in range(OFF):
            wait_gather(b)

        plsc.subcore_barrier()
        pltpu.sync_copy(acc.at[pl.ds(ss * RPT, RPT)],
                        out_hbm.at[cc, pl.ds(ss * RPT, RPT)])

    return agg


_agg1 = _make_agg(64, NCH_T, col_split=True, dtype=jnp.bfloat16)
_agg2 = _make_agg(64, NCH_W, col_split=False)

_DEGC = 16
_DG = 8


_REAL_LAST = E - (NW - 1) * EPW
_PADN = E_PAD - E


@functools.partial(
    pl.kernel,
    out_type=(
        jax.ShapeDtypeStruct((NC, N_PAD, _DEGC), jnp.float32),
        jax.ShapeDtypeStruct((E_PAD,), jnp.int32),
        jax.ShapeDtypeStruct((E_PAD,), jnp.int32),
    ),
    mesh=_mesh,
    compiler_params=_params,
    scratch_types=[
        pltpu.VMEM((EPW,), jnp.int32),
        pltpu.VMEM((EPW,), jnp.int32),
        pltpu.VMEM((K, _DEGC), jnp.float32),
        pltpu.VMEM_SHARED((N_PAD, _DEGC), jnp.float32),
        pltpu.SemaphoreType.DMA,
    ],
)
def _deg(edge_hbm, pads_hbm, ones_hbm, zeros_hbm,
         out_hbm, src_lin, dst_lin, src_v, dst_v, ones_v, acc, sem):
    cc = lax.axis_index("c")
    ss = lax.axis_index("s")
    wid = ss * NC + cc

    pltpu.sync_copy(zeros_hbm, acc.at[pl.ds(ss * RPT, RPT)])
    plsc.subcore_barrier()

    @pl.when(wid < NW - 1)
    def _():
        pltpu.sync_copy(edge_hbm.at[0, pl.ds(wid * EPW, EPW)], src_v)
        pltpu.sync_copy(edge_hbm.at[1, pl.ds(wid * EPW, EPW)], dst_v)

    @pl.when(wid == NW - 1)
    def _():
        base = (NW - 1) * EPW
        pltpu.sync_copy(edge_hbm.at[0, pl.ds(base, _REAL_LAST)],
                        src_v.at[pl.ds(0, _REAL_LAST)])
        pltpu.sync_copy(edge_hbm.at[1, pl.ds(base, _REAL_LAST)],
                        dst_v.at[pl.ds(0, _REAL_LAST)])
        pltpu.sync_copy(pads_hbm.at[0], src_v.at[pl.ds(_REAL_LAST, _PADN)])
        pltpu.sync_copy(pads_hbm.at[1], dst_v.at[pl.ds(_REAL_LAST, _PADN)])

    pltpu.sync_copy(src_v, src_lin.at[pl.ds(wid * EPW, EPW)])
    pltpu.sync_copy(dst_v, dst_lin.at[pl.ds(wid * EPW, EPW)])

    pltpu.sync_copy(ones_hbm, ones_v)

    def step(i, carry):
        for b in range(_DG):
            pltpu.async_copy(ones_v, acc.at[dst_v.at[_ds(i * _DG + b)]], sem,
                             add=True)
        for b in range(_DG):
            pltpu.make_async_copy(ones_v, acc.at[dst_v.at[_ds(0)]],
                                  sem).wait()
        return carry

    lax.fori_loop(0, NCH_W // _DG, step, 0)

    plsc.subcore_barrier()
    pltpu.sync_copy(acc.at[pl.ds(ss * RPT, RPT)],
                    out_hbm.at[cc, pl.ds(ss * RPT, RPT)])


def _prep_body(degp_ref, x_ref, xp2_ref, dinv_ref):
    deg = degp_ref[0, :, 0:1] + degp_ref[1, :, 0:1] + 1.0
    dinv = lax.rsqrt(deg)
    xp = (x_ref[...] * dinv).astype(jnp.bfloat16)
    xp2_ref[0] = xp[:, :64]
    xp2_ref[1] = xp[:, 64:]
    dinv_ref[...] = dinv


def _prep(degp, x):
    return pl.pallas_call(
        _prep_body,
        grid=(GRID,),
        in_specs=[
            pl.BlockSpec((NC, RBLK, _DEGC), lambda i: (0, i, 0)),
            pl.BlockSpec((RBLK, 128), lambda i: (i, 0)),
        ],
        out_specs=[
            pl.BlockSpec((NC, RBLK, 64), lambda i: (0, i, 0)),
            pl.BlockSpec((RBLK, 1), lambda i: (i, 0)),
        ],
        out_shape=[
            jax.ShapeDtypeStruct((NC, N, 64), jnp.bfloat16),
            jax.ShapeDtypeStruct((N, 1), jnp.float32),
        ],
    )(degp, x)


def _dense_body(p_ref, xp2_ref, dinv_ref, W1_ref, b1_ref, W2_ref, out_ref):
    p = jnp.concatenate([p_ref[0], p_ref[1]], axis=1).astype(jnp.float32)
    xp = jnp.concatenate([xp2_ref[0], xp2_ref[1]], axis=1).astype(jnp.float32)
    agg = p + xp
    t = (agg * dinv_ref[...]).astype(jnp.bfloat16)
    h1 = jnp.dot(t, W1_ref[...].astype(jnp.bfloat16),
                 preferred_element_type=jnp.float32) + b1_ref[...]
    h1 = jnp.maximum(h1, 0.0).astype(jnp.bfloat16)
    h2 = jnp.dot(h1, W2_ref[...].astype(jnp.bfloat16),
                 preferred_element_type=jnp.float32)
    out_ref[...] = h2 * dinv_ref[...]


def _dense(p, xp2, dinv, W1, b1, W2):
    return pl.pallas_call(
        _dense_body,
        grid=(GRID,),
        in_specs=[
            pl.BlockSpec((NC, RBLK, 64), lambda i: (0, i, 0)),
            pl.BlockSpec((NC, RBLK, 64), lambda i: (0, i, 0)),
            pl.BlockSpec((RBLK, 1), lambda i: (i, 0)),
            pl.BlockSpec((128, 256), lambda i: (0, 0)),
            pl.BlockSpec((1, 256), lambda i: (0, 0)),
            pl.BlockSpec((256, 64), lambda i: (0, 0)),
        ],
        out_specs=pl.BlockSpec((RBLK, 64), lambda i: (i, 0)),
        out_shape=jax.ShapeDtypeStruct((N, 64), jnp.float32),
    )(p, xp2, dinv, W1, b1, W2)


def _final_body(q_ref, hp_ref, dinv_ref, b2_ref, out_ref):
    out_ref[...] = ((q_ref[0] + q_ref[1] + hp_ref[...]) * dinv_ref[...]
                    + b2_ref[...])


def _final(q, hp, dinv, b2):
    return pl.pallas_call(
        _final_body,
        grid=(GRID,),
        in_specs=[
            pl.BlockSpec((NC, RBLK, 64), lambda i: (0, i, 0)),
            pl.BlockSpec((RBLK, 64), lambda i: (i, 0)),
            pl.BlockSpec((RBLK, 1), lambda i: (i, 0)),
            pl.BlockSpec((1, 64), lambda i: (0, 0)),
        ],
        out_specs=pl.BlockSpec((RBLK, 64), lambda i: (i, 0)),
        out_shape=jax.ShapeDtypeStruct((N, 64), jnp.float32),
    )(q, hp, dinv, b2)


@jax.jit
def _run(x, edge_index, W1, b1, W2, b2):
    spread = jnp.arange(_PADN, dtype=jnp.int32) % (N_PAD - N)
    pads = jnp.stack([spread, N + spread])

    degp, src_p, dst_p = _deg(edge_index.astype(jnp.int32), pads,
                              jnp.ones((K, _DEGC), jnp.float32),
                              jnp.zeros((RPT, _DEGC), jnp.float32))
    xp2, dinv = _prep(degp, x)
    p = _agg1(src_p, dst_p, xp2, jnp.zeros((RPT, 64), jnp.bfloat16))
    hp = _dense(p, xp2, dinv, W1, b1.reshape(1, -1), W2)
    q = _agg2(src_p, dst_p, hp, jnp.zeros((RPT, 64), jnp.float32))
    return _final(q, hp, dinv, b2.reshape(1, -1))


def kernel(x, edge_index, W1, b1, W2, b2):
    return _run(x, edge_index, W1, b1, W2, b2)


# speedup vs baseline: 1.2253x; 1.0814x over previous
import functools

import jax
import jax.numpy as jnp
from jax import lax
from jax.experimental import pallas as pl
from jax.experimental.pallas import tpu as pltpu
from jax.experimental.pallas import tpu_sc as plsc

N = 10000
E = 320000
NC = 2
NS = 16
NW = NC * NS
K = 128
E_PAD = 327680
NCH_T = E_PAD // NS // K
NCH_W = E_PAD // NW // K
EPT = E_PAD // NS
EPW = E_PAD // NW
D = 4
OFF = D // 2
N_PAD = 10240
RPT = N_PAD // NS
RBLK = 2000
GRID = N // RBLK

_mesh = plsc.VectorSubcoreMesh(core_axis_name="c", subcore_axis_name="s")
_params = pltpu.CompilerParams(use_tc_tiling_on_sc=False)


def _ds(j):
    return pl.ds(pl.multiple_of(j * K, K), K)


def _make_agg(CH, NCH, col_split, dtype=jnp.float32):
    EPB = NCH * K

    @functools.partial(
        pl.kernel,
        out_type=jax.ShapeDtypeStruct((NC, N_PAD, CH), dtype),
        mesh=_mesh,
        compiler_params=_params,
        scratch_types=[
            pltpu.VMEM((EPB,), jnp.int32),
            pltpu.VMEM((EPB,), jnp.int32),
            [pltpu.VMEM((K, CH), dtype)] * D,
            pltpu.VMEM_SHARED((N_PAD, CH), dtype),
            [pltpu.SemaphoreType.DMA] * D,
            [pltpu.SemaphoreType.DMA] * D,
        ],
    )
    def agg(src_hbm, dst_hbm, table_hbm, zeros_hbm, out_hbm,
            src_v, dst_v, bufs, acc, gsems, ssems):
        cc = lax.axis_index("c")
        ss = lax.axis_index("s")
        wid = ss if col_split else ss * NC + cc

        pltpu.sync_copy(zeros_hbm, acc.at[pl.ds(ss * RPT, RPT)])
        plsc.subcore_barrier()

        pltpu.sync_copy(src_hbm.at[pl.ds(wid * EPB, EPB)], src_v)
        pltpu.sync_copy(dst_hbm.at[pl.ds(wid * EPB, EPB)], dst_v)

        tab = table_hbm.at[cc] if col_split else table_hbm

        def wait_gather(b):
            pltpu.make_async_copy(tab.at[src_v.at[_ds(0)]], bufs[b],
                                  gsems[b]).wait()

        def wait_scatter(b):
            pltpu.make_async_copy(bufs[b], acc.at[dst_v.at[_ds(0)]],
                                  ssems[b]).wait()

        for b in range(OFF):
            pltpu.async_copy(tab.at[src_v.at[_ds(b)]], bufs[b], gsems[b])

        def block(i, carry):
            jb = i * D
            for b in range(D):
                j = jb + b
                b2 = (b + OFF) % D
                wait_gather(b)
                pltpu.async_copy(bufs[b], acc.at[dst_v.at[_ds(j)]], ssems[b],
                                 add=True)
                if b >= D - OFF:
                    wait_scatter(b2)
                else:
                    @pl.when(i > 0)
                    def _():
                        wait_scatter(b2)
                jn = lax.min(j + OFF, NCH - 1)
                pltpu.async_copy(tab.at[src_v.at[_ds(jn)]], bufs[b2],
                                 gsems[b2])
            return carry

        lax.fori_loop(0, NCH // D, block, 0)
        for b in range(OFF, D):
            wait_scatter(b)
        for b in range(OFF):
            wait_gather(b)

        plsc.subcore_barrier()
        pltpu.sync_copy(acc.at[pl.ds(ss * RPT, RPT)],
                        out_hbm.at[cc, pl.ds(ss * RPT, RPT)])

    return agg


_agg1 = _make_agg(64, NCH_T, col_split=True, dtype=jnp.bfloat16)
_agg2 = _make_agg(64, NCH_W, col_split=False, dtype=jnp.bfloat16)

_DEGC = 16
_DG = 8


_REAL_LAST = E - (NW - 1) * EPW
_PADN = E_PAD - E


@functools.partial(
    pl.kernel,
    out_type=(
        jax.ShapeDtypeStruct((NC, N_PAD, _DEGC), jnp.float32),
        jax.ShapeDtypeStruct((E_PAD,), jnp.int32),
        jax.ShapeDtypeStruct((E_PAD,), jnp.int32),
    ),
    mesh=_mesh,
    compiler_params=_params,
    scratch_types=[
        pltpu.VMEM((EPW,), jnp.int32),
        pltpu.VMEM((EPW,), jnp.int32),
        pltpu.VMEM((K, _DEGC), jnp.float32),
        pltpu.VMEM_SHARED((N_PAD, _DEGC), jnp.float32),
        pltpu.SemaphoreType.DMA,
    ],
)
def _deg(edge_hbm, pads_hbm, ones_hbm, zeros_hbm,
         out_hbm, src_lin, dst_lin, src_v, dst_v, ones_v, acc, sem):
    cc = lax.axis_index("c")
    ss = lax.axis_index("s")
    wid = ss * NC + cc

    pltpu.sync_copy(zeros_hbm, acc.at[pl.ds(ss * RPT, RPT)])
    plsc.subcore_barrier()

    @pl.when(wid < NW - 1)
    def _():
        pltpu.sync_copy(edge_hbm.at[0, pl.ds(wid * EPW, EPW)], src_v)
        pltpu.sync_copy(edge_hbm.at[1, pl.ds(wid * EPW, EPW)], dst_v)

    @pl.when(wid == NW - 1)
    def _():
        base = (NW - 1) * EPW
        pltpu.sync_copy(edge_hbm.at[0, pl.ds(base, _REAL_LAST)],
                        src_v.at[pl.ds(0, _REAL_LAST)])
        pltpu.sync_copy(edge_hbm.at[1, pl.ds(base, _REAL_LAST)],
                        dst_v.at[pl.ds(0, _REAL_LAST)])
        pltpu.sync_copy(pads_hbm.at[0], src_v.at[pl.ds(_REAL_LAST, _PADN)])
        pltpu.sync_copy(pads_hbm.at[1], dst_v.at[pl.ds(_REAL_LAST, _PADN)])

    pltpu.sync_copy(src_v, src_lin.at[pl.ds(wid * EPW, EPW)])
    pltpu.sync_copy(dst_v, dst_lin.at[pl.ds(wid * EPW, EPW)])

    pltpu.sync_copy(ones_hbm, ones_v)

    def step(i, carry):
        for b in range(_DG):
            pltpu.async_copy(ones_v, acc.at[dst_v.at[_ds(i * _DG + b)]], sem,
                             add=True)
        for b in range(_DG):
            pltpu.make_async_copy(ones_v, acc.at[dst_v.at[_ds(0)]],
                                  sem).wait()
        return carry

    lax.fori_loop(0, NCH_W // _DG, step, 0)

    plsc.subcore_barrier()
    pltpu.sync_copy(acc.at[pl.ds(ss * RPT, RPT)],
                    out_hbm.at[cc, pl.ds(ss * RPT, RPT)])


def _prep_body(degp_ref, x_ref, xp2_ref, dinv_ref):
    deg = degp_ref[0, :, 0:1] + degp_ref[1, :, 0:1] + 1.0
    dinv = lax.rsqrt(deg)
    xp = (x_ref[...] * dinv).astype(jnp.bfloat16)
    xp2_ref[0] = xp[:, :64]
    xp2_ref[1] = xp[:, 64:]
    dinv_ref[...] = dinv


def _prep(degp, x):
    return pl.pallas_call(
        _prep_body,
        grid=(GRID,),
        in_specs=[
            pl.BlockSpec((NC, RBLK, _DEGC), lambda i: (0, i, 0)),
            pl.BlockSpec((RBLK, 128), lambda i: (i, 0)),
        ],
        out_specs=[
            pl.BlockSpec((NC, RBLK, 64), lambda i: (0, i, 0)),
            pl.BlockSpec((RBLK, 1), lambda i: (i, 0)),
        ],
        out_shape=[
            jax.ShapeDtypeStruct((NC, N, 64), jnp.bfloat16),
            jax.ShapeDtypeStruct((N, 1), jnp.float32),
        ],
    )(degp, x)


def _dense_body(p_ref, xp2_ref, dinv_ref, W1_ref, b1_ref, W2_ref, out_ref):
    p = jnp.concatenate([p_ref[0], p_ref[1]], axis=1).astype(jnp.float32)
    xp = jnp.concatenate([xp2_ref[0], xp2_ref[1]], axis=1).astype(jnp.float32)
    agg = p + xp
    t = (agg * dinv_ref[...]).astype(jnp.bfloat16)
    h1 = jnp.dot(t, W1_ref[...].astype(jnp.bfloat16),
                 preferred_element_type=jnp.float32) + b1_ref[...]
    h1 = jnp.maximum(h1, 0.0).astype(jnp.bfloat16)
    h2 = jnp.dot(h1, W2_ref[...].astype(jnp.bfloat16),
                 preferred_element_type=jnp.float32)
    out_ref[...] = (h2 * dinv_ref[...]).astype(jnp.bfloat16)


def _dense(p, xp2, dinv, W1, b1, W2):
    return pl.pallas_call(
        _dense_body,
        grid=(GRID,),
        in_specs=[
            pl.BlockSpec((NC, RBLK, 64), lambda i: (0, i, 0)),
            pl.BlockSpec((NC, RBLK, 64), lambda i: (0, i, 0)),
            pl.BlockSpec((RBLK, 1), lambda i: (i, 0)),
            pl.BlockSpec((128, 256), lambda i: (0, 0)),
            pl.BlockSpec((1, 256), lambda i: (0, 0)),
            pl.BlockSpec((256, 64), lambda i: (0, 0)),
        ],
        out_specs=pl.BlockSpec((RBLK, 64), lambda i: (i, 0)),
        out_shape=jax.ShapeDtypeStruct((N, 64), jnp.bfloat16),
    )(p, xp2, dinv, W1, b1, W2)


def _final_body(q_ref, hp_ref, dinv_ref, b2_ref, out_ref):
    q = q_ref[0].astype(jnp.float32) + q_ref[1].astype(jnp.float32)
    out_ref[...] = ((q + hp_ref[...].astype(jnp.float32)) * dinv_ref[...]
                    + b2_ref[...])


def _final(q, hp, dinv, b2):
    return pl.pallas_call(
        _final_body,
        grid=(GRID,),
        in_specs=[
            pl.BlockSpec((NC, RBLK, 64), lambda i: (0, i, 0)),
            pl.BlockSpec((RBLK, 64), lambda i: (i, 0)),
            pl.BlockSpec((RBLK, 1), lambda i: (i, 0)),
            pl.BlockSpec((1, 64), lambda i: (0, 0)),
        ],
        out_specs=pl.BlockSpec((RBLK, 64), lambda i: (i, 0)),
        out_shape=jax.ShapeDtypeStruct((N, 64), jnp.float32),
    )(q, hp, dinv, b2)


@jax.jit
def _run(x, edge_index, W1, b1, W2, b2):
    spread = jnp.arange(_PADN, dtype=jnp.int32) % (N_PAD - N)
    pads = jnp.stack([spread, N + spread])

    degp, src_p, dst_p = _deg(edge_index.astype(jnp.int32), pads,
                              jnp.ones((K, _DEGC), jnp.float32),
                              jnp.zeros((RPT, _DEGC), jnp.float32))
    xp2, dinv = _prep(degp, x)
    p = _agg1(src_p, dst_p, xp2, jnp.zeros((RPT, 64), jnp.bfloat16))
    hp = _dense(p, xp2, dinv, W1, b1.reshape(1, -1), W2)
    q = _agg2(src_p, dst_p, hp, jnp.zeros((RPT, 64), jnp.bfloat16))
    return _final(q, hp, dinv, b2.reshape(1, -1))


def kernel(x, edge_index, W1, b1, W2, b2):
    return _run(x, edge_index, W1, b1, W2, b2)


# speedup vs baseline: 1.3755x; 1.1226x over previous
import functools

import jax
import jax.numpy as jnp
from jax import lax
from jax.experimental import pallas as pl
from jax.experimental.pallas import tpu as pltpu
from jax.experimental.pallas import tpu_sc as plsc

N = 10000
E = 320000
NC = 2
NS = 16
NW = NC * NS
K = 256
E_PAD = 327680
NCH_T = E_PAD // NS // K
NCH_W = E_PAD // NW // K
EPT = E_PAD // NS
EPW = E_PAD // NW
D = 4
OFF = D // 2
N_PAD = 10240
RPT = N_PAD // NS
RBLK = 2000
GRID = N // RBLK

_mesh = plsc.VectorSubcoreMesh(core_axis_name="c", subcore_axis_name="s")
_params = pltpu.CompilerParams(use_tc_tiling_on_sc=False)


def _ds(j):
    return pl.ds(pl.multiple_of(j * K, K), K)


def _make_agg(CH, NCH, col_split, dtype=jnp.float32):
    EPB = NCH * K

    @functools.partial(
        pl.kernel,
        out_type=jax.ShapeDtypeStruct((NC, N_PAD, CH), dtype),
        mesh=_mesh,
        compiler_params=_params,
        scratch_types=[
            pltpu.VMEM((EPB,), jnp.int32),
            pltpu.VMEM((EPB,), jnp.int32),
            [pltpu.VMEM((K, CH), dtype)] * D,
            pltpu.VMEM_SHARED((N_PAD, CH), dtype),
            [pltpu.SemaphoreType.DMA] * D,
            [pltpu.SemaphoreType.DMA] * D,
        ],
    )
    def agg(src_hbm, dst_hbm, table_hbm, zeros_hbm, out_hbm,
            src_v, dst_v, bufs, acc, gsems, ssems):
        cc = lax.axis_index("c")
        ss = lax.axis_index("s")
        wid = ss if col_split else ss * NC + cc

        pltpu.sync_copy(zeros_hbm, acc.at[pl.ds(ss * RPT, RPT)])
        plsc.subcore_barrier()

        pltpu.sync_copy(src_hbm.at[pl.ds(wid * EPB, EPB)], src_v)
        pltpu.sync_copy(dst_hbm.at[pl.ds(wid * EPB, EPB)], dst_v)

        tab = table_hbm.at[cc] if col_split else table_hbm

        def wait_gather(b):
            pltpu.make_async_copy(tab.at[src_v.at[_ds(0)]], bufs[b],
                                  gsems[b]).wait()

        def wait_scatter(b):
            pltpu.make_async_copy(bufs[b], acc.at[dst_v.at[_ds(0)]],
                                  ssems[b]).wait()

        for b in range(OFF):
            pltpu.async_copy(tab.at[src_v.at[_ds(b)]], bufs[b], gsems[b])

        def block(i, carry):
            jb = i * D
            for b in range(D):
                j = jb + b
                b2 = (b + OFF) % D
                wait_gather(b)
                pltpu.async_copy(bufs[b], acc.at[dst_v.at[_ds(j)]], ssems[b],
                                 add=True)
                if b >= D - OFF:
                    wait_scatter(b2)
                else:
                    @pl.when(i > 0)
                    def _():
                        wait_scatter(b2)
                jn = lax.min(j + OFF, NCH - 1)
                pltpu.async_copy(tab.at[src_v.at[_ds(jn)]], bufs[b2],
                                 gsems[b2])
            return carry

        lax.fori_loop(0, NCH // D, block, 0)
        for b in range(OFF, D):
            wait_scatter(b)
        for b in range(OFF):
            wait_gather(b)

        plsc.subcore_barrier()
        pltpu.sync_copy(acc.at[pl.ds(ss * RPT, RPT)],
                        out_hbm.at[cc, pl.ds(ss * RPT, RPT)])

    return agg


_agg1 = _make_agg(64, NCH_T, col_split=True, dtype=jnp.bfloat16)
_agg2 = _make_agg(64, NCH_W, col_split=False, dtype=jnp.bfloat16)

_DEGC = 8
_DG = 8


_REAL_LAST = E - (NW - 1) * EPW
_PADN = E_PAD - E


@functools.partial(
    pl.kernel,
    out_type=(
        jax.ShapeDtypeStruct((NC, N_PAD, _DEGC), jnp.float32),
        jax.ShapeDtypeStruct((E_PAD,), jnp.int32),
        jax.ShapeDtypeStruct((E_PAD,), jnp.int32),
    ),
    mesh=_mesh,
    compiler_params=_params,
    scratch_types=[
        pltpu.VMEM((EPW,), jnp.int32),
        pltpu.VMEM((EPW,), jnp.int32),
        pltpu.VMEM((K, _DEGC), jnp.float32),
        pltpu.VMEM_SHARED((N_PAD, _DEGC), jnp.float32),
        pltpu.SemaphoreType.DMA,
    ],
)
def _deg(edge_hbm, pads_hbm, ones_hbm, zeros_hbm,
         out_hbm, src_lin, dst_lin, src_v, dst_v, ones_v, acc, sem):
    cc = lax.axis_index("c")
    ss = lax.axis_index("s")
    wid = ss * NC + cc

    pltpu.sync_copy(zeros_hbm, acc.at[pl.ds(ss * RPT, RPT)])
    plsc.subcore_barrier()

    @pl.when(wid < NW - 1)
    def _():
        pltpu.sync_copy(edge_hbm.at[0, pl.ds(wid * EPW, EPW)], src_v)
        pltpu.sync_copy(edge_hbm.at[1, pl.ds(wid * EPW, EPW)], dst_v)

    @pl.when(wid == NW - 1)
    def _():
        base = (NW - 1) * EPW
        pltpu.sync_copy(edge_hbm.at[0, pl.ds(base, _REAL_LAST)],
                        src_v.at[pl.ds(0, _REAL_LAST)])
        pltpu.sync_copy(edge_hbm.at[1, pl.ds(base, _REAL_LAST)],
                        dst_v.at[pl.ds(0, _REAL_LAST)])
        pltpu.sync_copy(pads_hbm.at[0], src_v.at[pl.ds(_REAL_LAST, _PADN)])
        pltpu.sync_copy(pads_hbm.at[1], dst_v.at[pl.ds(_REAL_LAST, _PADN)])

    pltpu.sync_copy(src_v, src_lin.at[pl.ds(wid * EPW, EPW)])
    pltpu.sync_copy(dst_v, dst_lin.at[pl.ds(wid * EPW, EPW)])

    pltpu.sync_copy(ones_hbm, ones_v)

    def step(i, carry):
        for b in range(_DG):
            pltpu.async_copy(ones_v, acc.at[dst_v.at[_ds(i * _DG + b)]], sem,
                             add=True)
        for b in range(_DG):
            pltpu.make_async_copy(ones_v, acc.at[dst_v.at[_ds(0)]],
                                  sem).wait()
        return carry

    lax.fori_loop(0, NCH_W // _DG, step, 0)

    plsc.subcore_barrier()
    pltpu.sync_copy(acc.at[pl.ds(ss * RPT, RPT)],
                    out_hbm.at[cc, pl.ds(ss * RPT, RPT)])


def _prep_body(degp_ref, x_ref, xp2_ref, dinv_ref):
    deg = degp_ref[0, :, 0:1] + degp_ref[1, :, 0:1] + 1.0
    dinv = lax.rsqrt(deg)
    xp = (x_ref[...] * dinv).astype(jnp.bfloat16)
    xp2_ref[0] = xp[:, :64]
    xp2_ref[1] = xp[:, 64:]
    dinv_ref[...] = dinv


def _prep(degp, x):
    return pl.pallas_call(
        _prep_body,
        grid=(GRID,),
        in_specs=[
            pl.BlockSpec((NC, RBLK, _DEGC), lambda i: (0, i, 0)),
            pl.BlockSpec((RBLK, 128), lambda i: (i, 0)),
        ],
        out_specs=[
            pl.BlockSpec((NC, RBLK, 64), lambda i: (0, i, 0)),
            pl.BlockSpec((RBLK, 1), lambda i: (i, 0)),
        ],
        out_shape=[
            jax.ShapeDtypeStruct((NC, N, 64), jnp.bfloat16),
            jax.ShapeDtypeStruct((N, 1), jnp.float32),
        ],
    )(degp, x)


def _dense_body(p_ref, xp2_ref, dinv_ref, W1_ref, b1_ref, W2_ref, out_ref):
    p = jnp.concatenate([p_ref[0], p_ref[1]], axis=1).astype(jnp.float32)
    xp = jnp.concatenate([xp2_ref[0], xp2_ref[1]], axis=1).astype(jnp.float32)
    agg = p + xp
    t = (agg * dinv_ref[...]).astype(jnp.bfloat16)
    h1 = jnp.dot(t, W1_ref[...],
                 preferred_element_type=jnp.float32) + b1_ref[...]
    h1 = jnp.maximum(h1, 0.0).astype(jnp.bfloat16)
    h2 = jnp.dot(h1, W2_ref[...],
                 preferred_element_type=jnp.float32)
    out_ref[...] = (h2 * dinv_ref[...]).astype(jnp.bfloat16)


def _dense(p, xp2, dinv, W1, b1, W2):
    return pl.pallas_call(
        _dense_body,
        grid=(GRID,),
        in_specs=[
            pl.BlockSpec((NC, RBLK, 64), lambda i: (0, i, 0)),
            pl.BlockSpec((NC, RBLK, 64), lambda i: (0, i, 0)),
            pl.BlockSpec((RBLK, 1), lambda i: (i, 0)),
            pl.BlockSpec((128, 256), lambda i: (0, 0)),
            pl.BlockSpec((1, 256), lambda i: (0, 0)),
            pl.BlockSpec((256, 64), lambda i: (0, 0)),
        ],
        out_specs=pl.BlockSpec((RBLK, 64), lambda i: (i, 0)),
        out_shape=jax.ShapeDtypeStruct((N, 64), jnp.bfloat16),
    )(p, xp2, dinv, W1, b1, W2)


def _final_body(q_ref, hp_ref, dinv_ref, b2_ref, out_ref):
    q = q_ref[0].astype(jnp.float32) + q_ref[1].astype(jnp.float32)
    out_ref[...] = ((q + hp_ref[...].astype(jnp.float32)) * dinv_ref[...]
                    + b2_ref[...])


def _final(q, hp, dinv, b2):
    return pl.pallas_call(
        _final_body,
        grid=(GRID,),
        in_specs=[
            pl.BlockSpec((NC, RBLK, 64), lambda i: (0, i, 0)),
            pl.BlockSpec((RBLK, 64), lambda i: (i, 0)),
            pl.BlockSpec((RBLK, 1), lambda i: (i, 0)),
            pl.BlockSpec((1, 64), lambda i: (0, 0)),
        ],
        out_specs=pl.BlockSpec((RBLK, 64), lambda i: (i, 0)),
        out_shape=jax.ShapeDtypeStruct((N, 64), jnp.float32),
    )(q, hp, dinv, b2)


@jax.jit
def _run(x, edge_index, W1, b1, W2, b2):
    spread = jnp.arange(_PADN, dtype=jnp.int32) % (N_PAD - N)
    pads = jnp.stack([spread, N + spread])

    degp, src_p, dst_p = _deg(edge_index.astype(jnp.int32), pads,
                              jnp.ones((K, _DEGC), jnp.float32),
                              jnp.zeros((RPT, _DEGC), jnp.float32))
    xp2, dinv = _prep(degp, x)
    p = _agg1(src_p, dst_p, xp2, jnp.zeros((RPT, 64), jnp.bfloat16))
    hp = _dense(p, xp2, dinv, W1.astype(jnp.bfloat16), b1.reshape(1, -1),
                W2.astype(jnp.bfloat16))
    q = _agg2(src_p, dst_p, hp, jnp.zeros((RPT, 64), jnp.bfloat16))
    return _final(q, hp, dinv, b2.reshape(1, -1))


def kernel(x, edge_index, W1, b1, W2, b2):
    return _run(x, edge_index, W1, b1, W2, b2)


# speedup vs baseline: 1.4170x; 1.0302x over previous
import functools

import jax
import jax.numpy as jnp
from jax import lax
from jax.experimental import pallas as pl
from jax.experimental.pallas import tpu as pltpu
from jax.experimental.pallas import tpu_sc as plsc

N = 10000
E = 320000
NC = 2
NS = 16
NW = NC * NS
K = 512
E_PAD = 327680
NCH_T = E_PAD // NS // K
NCH_W = E_PAD // NW // K
EPT = E_PAD // NS
EPW = E_PAD // NW
D = 4
OFF = D // 2
N_PAD = 10240
RPT = N_PAD // NS
RBLK = 2000
GRID = N // RBLK

_mesh = plsc.VectorSubcoreMesh(core_axis_name="c", subcore_axis_name="s")
_params = pltpu.CompilerParams(use_tc_tiling_on_sc=False)


def _ds(j):
    return pl.ds(pl.multiple_of(j * K, K), K)


def _make_agg(CH, NCH, col_split, dtype=jnp.float32):
    EPB = NCH * K

    @functools.partial(
        pl.kernel,
        out_type=jax.ShapeDtypeStruct((NC, N_PAD, CH), dtype),
        mesh=_mesh,
        compiler_params=_params,
        scratch_types=[
            pltpu.VMEM((EPB,), jnp.int32),
            pltpu.VMEM((EPB,), jnp.int32),
            [pltpu.VMEM((K, CH), dtype)] * D,
            pltpu.VMEM_SHARED((N_PAD, CH), dtype),
            [pltpu.SemaphoreType.DMA] * D,
            [pltpu.SemaphoreType.DMA] * D,
        ],
    )
    def agg(src_hbm, dst_hbm, table_hbm, zeros_hbm, out_hbm,
            src_v, dst_v, bufs, acc, gsems, ssems):
        cc = lax.axis_index("c")
        ss = lax.axis_index("s")
        wid = ss if col_split else ss * NC + cc

        pltpu.sync_copy(zeros_hbm, acc.at[pl.ds(ss * RPT, RPT)])
        plsc.subcore_barrier()

        pltpu.sync_copy(src_hbm.at[pl.ds(wid * EPB, EPB)], src_v)
        pltpu.sync_copy(dst_hbm.at[pl.ds(wid * EPB, EPB)], dst_v)

        tab = table_hbm.at[cc] if col_split else table_hbm

        def wait_gather(b):
            pltpu.make_async_copy(tab.at[src_v.at[_ds(0)]], bufs[b],
                                  gsems[b]).wait()

        def wait_scatter(b):
            pltpu.make_async_copy(bufs[b], acc.at[dst_v.at[_ds(0)]],
                                  ssems[b]).wait()

        for b in range(OFF):
            pltpu.async_copy(tab.at[src_v.at[_ds(b)]], bufs[b], gsems[b])

        def block(i, carry):
            jb = i * D
            for b in range(D):
                j = jb + b
                b2 = (b + OFF) % D
                wait_gather(b)
                pltpu.async_copy(bufs[b], acc.at[dst_v.at[_ds(j)]], ssems[b],
                                 add=True)
                if b >= D - OFF:
                    wait_scatter(b2)
                else:
                    @pl.when(i > 0)
                    def _():
                        wait_scatter(b2)
                jn = lax.min(j + OFF, NCH - 1)
                pltpu.async_copy(tab.at[src_v.at[_ds(jn)]], bufs[b2],
                                 gsems[b2])
            return carry

        lax.fori_loop(0, NCH // D, block, 0)
        for b in range(OFF, D):
            wait_scatter(b)
        for b in range(OFF):
            wait_gather(b)

        plsc.subcore_barrier()
        pltpu.sync_copy(acc.at[pl.ds(ss * RPT, RPT)],
                        out_hbm.at[cc, pl.ds(ss * RPT, RPT)])

    return agg


_agg1 = _make_agg(64, NCH_T, col_split=True, dtype=jnp.bfloat16)
_agg2 = _make_agg(64, NCH_W, col_split=False, dtype=jnp.bfloat16)

_DEGC = 8
_DG = 4


_REAL_LAST = E - (NW - 1) * EPW
_PADN = E_PAD - E


@functools.partial(
    pl.kernel,
    out_type=(
        jax.ShapeDtypeStruct((NC, N_PAD, _DEGC), jnp.float32),
        jax.ShapeDtypeStruct((E_PAD,), jnp.int32),
        jax.ShapeDtypeStruct((E_PAD,), jnp.int32),
    ),
    mesh=_mesh,
    compiler_params=_params,
    scratch_types=[
        pltpu.VMEM((EPW,), jnp.int32),
        pltpu.VMEM((EPW,), jnp.int32),
        pltpu.VMEM((K, _DEGC), jnp.float32),
        pltpu.VMEM_SHARED((N_PAD, _DEGC), jnp.float32),
        pltpu.SemaphoreType.DMA,
    ],
)
def _deg(edge_hbm, pads_hbm, ones_hbm, zeros_hbm,
         out_hbm, src_lin, dst_lin, src_v, dst_v, ones_v, acc, sem):
    cc = lax.axis_index("c")
    ss = lax.axis_index("s")
    wid = ss * NC + cc

    pltpu.sync_copy(zeros_hbm, acc.at[pl.ds(ss * RPT, RPT)])
    plsc.subcore_barrier()

    @pl.when(wid < NW - 1)
    def _():
        pltpu.sync_copy(edge_hbm.at[0, pl.ds(wid * EPW, EPW)], src_v)
        pltpu.sync_copy(edge_hbm.at[1, pl.ds(wid * EPW, EPW)], dst_v)

    @pl.when(wid == NW - 1)
    def _():
        base = (NW - 1) * EPW
        pltpu.sync_copy(edge_hbm.at[0, pl.ds(base, _REAL_LAST)],
                        src_v.at[pl.ds(0, _REAL_LAST)])
        pltpu.sync_copy(edge_hbm.at[1, pl.ds(base, _REAL_LAST)],
                        dst_v.at[pl.ds(0, _REAL_LAST)])
        pltpu.sync_copy(pads_hbm.at[0], src_v.at[pl.ds(_REAL_LAST, _PADN)])
        pltpu.sync_copy(pads_hbm.at[1], dst_v.at[pl.ds(_REAL_LAST, _PADN)])

    pltpu.sync_copy(src_v, src_lin.at[pl.ds(wid * EPW, EPW)])
    pltpu.sync_copy(dst_v, dst_lin.at[pl.ds(wid * EPW, EPW)])

    pltpu.sync_copy(ones_hbm, ones_v)

    def step(i, carry):
        for b in range(_DG):
            pltpu.async_copy(ones_v, acc.at[dst_v.at[_ds(i * _DG + b)]], sem,
                             add=True)
        for b in range(_DG):
            pltpu.make_async_copy(ones_v, acc.at[dst_v.at[_ds(0)]],
                                  sem).wait()
        return carry

    lax.fori_loop(0, NCH_W // _DG, step, 0)

    plsc.subcore_barrier()
    pltpu.sync_copy(acc.at[pl.ds(ss * RPT, RPT)],
                    out_hbm.at[cc, pl.ds(ss * RPT, RPT)])


def _prep_body(degp_ref, x_ref, xp2_ref, dinv_ref):
    deg = degp_ref[0, :, 0:1] + degp_ref[1, :, 0:1] + 1.0
    dinv = lax.rsqrt(deg)
    xp = (x_ref[...] * dinv).astype(jnp.bfloat16)
    xp2_ref[0] = xp[:, :64]
    xp2_ref[1] = xp[:, 64:]
    dinv_ref[...] = dinv


def _prep(degp, x):
    return pl.pallas_call(
        _prep_body,
        grid=(GRID,),
        in_specs=[
            pl.BlockSpec((NC, RBLK, _DEGC), lambda i: (0, i, 0)),
            pl.BlockSpec((RBLK, 128), lambda i: (i, 0)),
        ],
        out_specs=[
            pl.BlockSpec((NC, RBLK, 64), lambda i: (0, i, 0)),
            pl.BlockSpec((RBLK, 1), lambda i: (i, 0)),
        ],
        out_shape=[
            jax.ShapeDtypeStruct((NC, N, 64), jnp.bfloat16),
            jax.ShapeDtypeStruct((N, 1), jnp.float32),
        ],
    )(degp, x)


def _dense_body(p_ref, xp2_ref, dinv_ref, W1_ref, b1_ref, W2_ref, out_ref):
    p = jnp.concatenate([p_ref[0], p_ref[1]], axis=1).astype(jnp.float32)
    xp = jnp.concatenate([xp2_ref[0], xp2_ref[1]], axis=1).astype(jnp.float32)
    agg = p + xp
    t = (agg * dinv_ref[...]).astype(jnp.bfloat16)
    h1 = jnp.dot(t, W1_ref[...],
                 preferred_element_type=jnp.float32) + b1_ref[...]
    h1 = jnp.maximum(h1, 0.0).astype(jnp.bfloat16)
    h2 = jnp.dot(h1, W2_ref[...],
                 preferred_element_type=jnp.float32)
    out_ref[...] = (h2 * dinv_ref[...]).astype(jnp.bfloat16)


def _dense(p, xp2, dinv, W1, b1, W2):
    return pl.pallas_call(
        _dense_body,
        grid=(GRID,),
        in_specs=[
            pl.BlockSpec((NC, RBLK, 64), lambda i: (0, i, 0)),
            pl.BlockSpec((NC, RBLK, 64), lambda i: (0, i, 0)),
            pl.BlockSpec((RBLK, 1), lambda i: (i, 0)),
            pl.BlockSpec((128, 256), lambda i: (0, 0)),
            pl.BlockSpec((1, 256), lambda i: (0, 0)),
            pl.BlockSpec((256, 64), lambda i: (0, 0)),
        ],
        out_specs=pl.BlockSpec((RBLK, 64), lambda i: (i, 0)),
        out_shape=jax.ShapeDtypeStruct((N, 64), jnp.bfloat16),
    )(p, xp2, dinv, W1, b1, W2)


def _final_body(q_ref, hp_ref, dinv_ref, b2_ref, out_ref):
    q = q_ref[0].astype(jnp.float32) + q_ref[1].astype(jnp.float32)
    out_ref[...] = ((q + hp_ref[...].astype(jnp.float32)) * dinv_ref[...]
                    + b2_ref[...])


def _final(q, hp, dinv, b2):
    return pl.pallas_call(
        _final_body,
        grid=(GRID,),
        in_specs=[
            pl.BlockSpec((NC, RBLK, 64), lambda i: (0, i, 0)),
            pl.BlockSpec((RBLK, 64), lambda i: (i, 0)),
            pl.BlockSpec((RBLK, 1), lambda i: (i, 0)),
            pl.BlockSpec((1, 64), lambda i: (0, 0)),
        ],
        out_specs=pl.BlockSpec((RBLK, 64), lambda i: (i, 0)),
        out_shape=jax.ShapeDtypeStruct((N, 64), jnp.float32),
    )(q, hp, dinv, b2)


@jax.jit
def _run(x, edge_index, W1, b1, W2, b2):
    spread = jnp.arange(_PADN, dtype=jnp.int32) % (N_PAD - N)
    pads = jnp.stack([spread, N + spread])

    degp, src_p, dst_p = _deg(edge_index.astype(jnp.int32), pads,
                              jnp.ones((K, _DEGC), jnp.float32),
                              jnp.zeros((RPT, _DEGC), jnp.float32))
    xp2, dinv = _prep(degp, x)
    p = _agg1(src_p, dst_p, xp2, jnp.zeros((RPT, 64), jnp.bfloat16))
    hp = _dense(p, xp2, dinv, W1.astype(jnp.bfloat16), b1.reshape(1, -1),
                W2.astype(jnp.bfloat16))
    q = _agg2(src_p, dst_p, hp, jnp.zeros((RPT, 64), jnp.bfloat16))
    return _final(q, hp, dinv, b2.reshape(1, -1))


def kernel(x, edge_index, W1, b1, W2, b2):
    return _run(x, edge_index, W1, b1, W2, b2)
